# Initial kernel scaffold; baseline (speedup 1.0000x reference)
import jax
import jax.numpy as jnp
from jax.experimental import pallas as pl


def kernel(etype, rid, att_rc, att_rp, W_type, W_rid, W_rc, b_rc, W_rp, b_rp):
    raise NotImplementedError("write your pallas kernel here")


# speedup vs baseline: 1.2676x; 1.2676x over previous
import functools

import jax
import jax.numpy as jnp
from jax import lax
from jax.experimental import pallas as pl
from jax.experimental.pallas import tpu as pltpu
from jax.experimental.pallas import tpu_sc as plsc

_E = 320000
_HID = 128
_NTYPE = 35
_NRID = 9
_NC = 2
_NS = 16
_NW = _NC * _NS
_PER_W = _E // _NW
_NB = 80
_CHUNKS = _PER_W // _NB
_L = 16


def _splat(vec, lane):
    return vec[jnp.full((_L,), lane, jnp.int32)]


def _sc_body(combo_h, et_h, ri_h, rc_h, rp_h, w_h, out_h,
             et_v, ri_v, idx_v, rc_v, rp_v, w_v, rows_v, sem):
    wid = lax.axis_index("s") * _NC + lax.axis_index("c")
    base = wid * _PER_W

    pltpu.sync_copy(w_h, w_v)
    wv = [[w_v[pl.ds(128 * k + 16 * q, _L)] for q in range(8)]
          for k in range(5)]

    def chunk(g, carry):
        cb = base + g * _NB
        pltpu.sync_copy(et_h.at[pl.ds(cb, _NB)], et_v)
        pltpu.sync_copy(ri_h.at[pl.ds(cb, _NB)], ri_v)
        pltpu.sync_copy(rc_h.at[pl.ds(cb * 2, _NB * 2)], rc_v)
        pltpu.sync_copy(rp_h.at[pl.ds(cb * 3, _NB * 3)], rp_v)
        for s in range(_NB // _L):
            et = et_v[pl.ds(_L * s, _L)]
            ri = ri_v[pl.ds(_L * s, _L)]
            idx_v[pl.ds(_L * s, _L)] = et * _NRID + ri
        pltpu.async_copy(combo_h.at[idx_v], rows_v, sem).wait()
        for gg in range(_NB // _L):
            rcg = [rc_v[pl.ds(32 * gg, _L)], rc_v[pl.ds(32 * gg + 16, _L)]]
            rpg = [rp_v[pl.ds(48 * gg, _L)], rp_v[pl.ds(48 * gg + 16, _L)],
                   rp_v[pl.ds(48 * gg + 32, _L)]]
            for j in range(_L):
                e = gg * _L + j
                a0 = _splat(rcg[(2 * j) // _L], (2 * j) % _L)
                a1 = _splat(rcg[(2 * j + 1) // _L], (2 * j + 1) % _L)
                p0 = _splat(rpg[(3 * j) // _L], (3 * j) % _L)
                p1 = _splat(rpg[(3 * j + 1) // _L], (3 * j + 1) % _L)
                p2 = _splat(rpg[(3 * j + 2) // _L], (3 * j + 2) % _L)
                for q in range(8):
                    r = rows_v[e, pl.ds(_L * q, _L)]
                    acc = (r + a0 * wv[0][q] + a1 * wv[1][q]
                           + p0 * wv[2][q] + p1 * wv[3][q] + p2 * wv[4][q])
                    rows_v[e, pl.ds(_L * q, _L)] = acc
        pltpu.sync_copy(rows_v, out_h.at[pl.ds(cb, _NB)])
        return carry

    lax.fori_loop(0, _CHUNKS, chunk, 0)


_sc_call = functools.partial(
    pl.kernel,
    out_type=jax.ShapeDtypeStruct((_E, _HID), jnp.float32),
    mesh=plsc.VectorSubcoreMesh(
        core_axis_name="c", subcore_axis_name="s",
        num_cores=_NC, num_subcores=_NS),
    scratch_types=[
        pltpu.VMEM((_NB,), jnp.int32),
        pltpu.VMEM((_NB,), jnp.int32),
        pltpu.VMEM((_NB,), jnp.int32),
        pltpu.VMEM((_NB * 2,), jnp.float32),
        pltpu.VMEM((_NB * 3,), jnp.float32),
        pltpu.VMEM((5 * _HID,), jnp.float32),
        pltpu.VMEM((_NB, _HID), jnp.float32),
        pltpu.SemaphoreType.DMA,
    ],
)(_sc_body)


@jax.jit
def kernel(etype, rid, att_rc, att_rp, W_type, W_rid, W_rc, b_rc, W_rp, b_rp):
    etype = etype.astype(jnp.int32)
    rid = rid.astype(jnp.int32)
    combo = ((W_type[:, None, :] + W_rid[None, :, :])
             .reshape(_NTYPE * _NRID, _HID) + b_rc + b_rp)
    wcat = jnp.concatenate([W_rc.T, W_rp.T], axis=0).reshape(-1)
    rc_flat = att_rc.reshape(-1)
    rp_flat = att_rp.astype(jnp.float32).reshape(-1)
    return _sc_call(combo, etype, rid, rc_flat, rp_flat, wcat)


# speedup vs baseline: 1.4807x; 1.1681x over previous
import functools

import jax
import jax.numpy as jnp
from jax import lax
from jax.experimental import pallas as pl
from jax.experimental.pallas import tpu as pltpu
from jax.experimental.pallas import tpu_sc as plsc

_E = 320000
_HID = 128
_NTYPE = 35
_NRID = 9
_NC = 2
_NS = 16
_NW = _NC * _NS
_PER_W = _E // _NW
_NB = 80
_CHUNKS = _PER_W // _NB
_L = 16


def _splat(vec, lane):
    return vec[jnp.full((_L,), lane, jnp.int32)]


def _sc_body(combo_h, et_h, ri_h, a0_h, a1_h, a2_h, a3_h, a4_h, w_h, out_h,
             et_v, ri_v, idx_v, a_v, w_v, rows_v, sem):
    wid = lax.axis_index("s") * _NC + lax.axis_index("c")
    base = wid * _PER_W

    pltpu.sync_copy(w_h, w_v)
    wv = [[w_v[pl.ds(128 * k + 16 * q, _L)] for q in range(8)]
          for k in range(5)]

    def chunk(g, carry):
        cb = base + g * _NB
        pltpu.sync_copy(et_h.at[pl.ds(cb, _NB)], et_v)
        pltpu.sync_copy(ri_h.at[pl.ds(cb, _NB)], ri_v)
        for k, ah in enumerate((a0_h, a1_h, a2_h, a3_h, a4_h)):
            pltpu.sync_copy(ah.at[pl.ds(cb, _NB)], a_v.at[k])
        for s in range(_NB // _L):
            et = et_v[pl.ds(_L * s, _L)]
            ri = ri_v[pl.ds(_L * s, _L)]
            idx_v[pl.ds(_L * s, _L)] = et * _NRID + ri
        pltpu.async_copy(combo_h.at[idx_v], rows_v, sem).wait()
        for gg in range(_NB // _L):
            av = [a_v[k, pl.ds(_L * gg, _L)] for k in range(5)]
            for j in range(_L):
                e = gg * _L + j
                c = [_splat(av[k], j) for k in range(5)]
                for q in range(8):
                    r = rows_v[e, pl.ds(_L * q, _L)]
                    acc = (r + c[0] * wv[0][q] + c[1] * wv[1][q]
                           + c[2] * wv[2][q] + c[3] * wv[3][q]
                           + c[4] * wv[4][q])
                    rows_v[e, pl.ds(_L * q, _L)] = acc
        pltpu.sync_copy(rows_v, out_h.at[pl.ds(cb, _NB)])
        return carry

    lax.fori_loop(0, _CHUNKS, chunk, 0)


_sc_call = functools.partial(
    pl.kernel,
    out_type=jax.ShapeDtypeStruct((_E, _HID), jnp.float32),
    mesh=plsc.VectorSubcoreMesh(
        core_axis_name="c", subcore_axis_name="s",
        num_cores=_NC, num_subcores=_NS),
    scratch_types=[
        pltpu.VMEM((_NB,), jnp.int32),
        pltpu.VMEM((_NB,), jnp.int32),
        pltpu.VMEM((_NB,), jnp.int32),
        pltpu.VMEM((5, _NB), jnp.float32),
        pltpu.VMEM((5 * _HID,), jnp.float32),
        pltpu.VMEM((_NB, _HID), jnp.float32),
        pltpu.SemaphoreType.DMA,
    ],
)(_sc_body)


@jax.jit
def kernel(etype, rid, att_rc, att_rp, W_type, W_rid, W_rc, b_rc, W_rp, b_rp):
    etype = etype.astype(jnp.int32)
    rid = rid.astype(jnp.int32)
    combo = ((W_type[:, None, :] + W_rid[None, :, :])
             .reshape(_NTYPE * _NRID, _HID) + b_rc + b_rp)
    wcat = jnp.concatenate([W_rc.T, W_rp.T], axis=0).reshape(-1)
    att_rp = att_rp.astype(jnp.float32)
    cols = [att_rc[:, 0], att_rc[:, 1],
            att_rp[:, 0], att_rp[:, 1], att_rp[:, 2]]
    return _sc_call(combo, etype, rid, *cols, wcat)


# speedup vs baseline: 3.2451x; 2.1916x over previous
import functools

import jax
import jax.numpy as jnp
from jax import lax
from jax.experimental import pallas as pl
from jax.experimental.pallas import tpu as pltpu
from jax.experimental.pallas import tpu_sc as plsc

_E = 320000
_HID = 128
_NTYPE = 35
_NRID = 9
_NC = 2
_NS = 16
_NW = _NC * _NS
_PER_W = _E // _NW
_NB = 80
_CHUNKS = _PER_W // _NB
_L = 16
_PKI = 2 * _NB
_PKF = 5 * _NB


def _splat(vec, lane):
    return vec[jnp.full((_L,), lane, jnp.int32)]


def _sc_body(combo_h, pki_h, pkf_h, w_h, out_h,
             pki0, pki1, pkf0, pkf1, idx0, idx1, rows0, rows1, w_v,
             is0, is1, gs0, gs1, os0, os1):
    wid = lax.axis_index("s") * _NC + lax.axis_index("c")
    base = wid * _PER_W

    pltpu.sync_copy(w_h, w_v)
    wv = [[w_v[pl.ds(128 * k + 16 * q, _L)] for q in range(8)]
          for k in range(5)]
    pki = (pki0, pki1)
    pkf = (pkf0, pkf1)
    idxv = (idx0, idx1)
    rows = (rows0, rows1)
    isem = (is0, is1)
    gsem = (gs0, gs1)
    osem = (os0, os1)

    def in_copies(c, d):
        gc = wid * _CHUNKS + c
        return (
            pltpu.make_async_copy(
                pki_h.at[pl.ds(gc * _PKI, _PKI)], pki[d], isem[d]),
            pltpu.make_async_copy(
                pkf_h.at[pl.ds(gc * _PKF, _PKF)], pkf[d], isem[d]),
        )

    def in_start(c, d):
        for cp in in_copies(c, d):
            cp.start()

    def in_wait(c, d):
        for cp in in_copies(c, d):
            cp.wait()

    def gather_copy(d):
        return pltpu.make_async_copy(combo_h.at[idxv[d]], rows[d], gsem[d])

    def out_copy(c, d):
        cb = base + c * _NB
        return pltpu.make_async_copy(
            rows[d], out_h.at[pl.ds(cb, _NB)], osem[d])

    def compute_idx(d):
        for s in range(_NB // _L):
            et = pki[d][pl.ds(_L * s, _L)]
            ri = pki[d][pl.ds(_NB + _L * s, _L)]
            idxv[d][pl.ds(_L * s, _L)] = et * _NRID + ri

    def fma(d):
        def grp(gg, carry):
            av = [pkf[d][pl.ds(k * _NB + _L * gg, _L)] for k in range(5)]
            eb = gg * _L
            for j in range(_L):
                cf = [_splat(av[k], j) for k in range(5)]
                for q in range(8):
                    r = rows[d][eb + j, pl.ds(_L * q, _L)]
                    acc = (r + cf[0] * wv[0][q] + cf[1] * wv[1][q]
                           + cf[2] * wv[2][q] + cf[3] * wv[3][q]
                           + cf[4] * wv[4][q])
                    rows[d][eb + j, pl.ds(_L * q, _L)] = acc
            return carry
        lax.fori_loop(0, _NB // _L, grp, 0)

    def do_step(c, d, first=False, fire_gather=True, fire_in=True):
        dn = 1 - d
        if fire_gather:
            in_wait(c + 1, dn)
            compute_idx(dn)
            if not first:
                out_copy(c - 1, dn).wait()
            gather_copy(dn).start()
        gather_copy(d).wait()
        fma(d)
        out_copy(c, d).start()
        if fire_in:
            in_start(c + 2, d)

    in_start(0, 0)
    in_start(1, 1)
    in_wait(0, 0)
    compute_idx(0)
    gather_copy(0).start()

    do_step(0, 0, first=True)

    def pair(i, carry):
        c = 2 * i
        do_step(c, 0)
        do_step(c + 1, 1)
        return carry

    do_step(1, 1)
    lax.fori_loop(1, (_CHUNKS - 3) // 2, pair, 0)
    do_step(_CHUNKS - 3, 0)
    do_step(_CHUNKS - 2, 1, fire_in=False)
    do_step(_CHUNKS - 1, 0, fire_gather=False, fire_in=False)

    out_copy(_CHUNKS - 2, 1).wait()
    out_copy(_CHUNKS - 1, 0).wait()


_sc_call = functools.partial(
    pl.kernel,
    out_type=jax.ShapeDtypeStruct((_E, _HID), jnp.float32),
    mesh=plsc.VectorSubcoreMesh(
        core_axis_name="c", subcore_axis_name="s",
        num_cores=_NC, num_subcores=_NS),
    scratch_types=[
        pltpu.VMEM((_PKI,), jnp.int32),
        pltpu.VMEM((_PKI,), jnp.int32),
        pltpu.VMEM((_PKF,), jnp.float32),
        pltpu.VMEM((_PKF,), jnp.float32),
        pltpu.VMEM((_NB,), jnp.int32),
        pltpu.VMEM((_NB,), jnp.int32),
        pltpu.VMEM((_NB, _HID), jnp.float32),
        pltpu.VMEM((_NB, _HID), jnp.float32),
        pltpu.VMEM((5 * _HID,), jnp.float32),
        pltpu.SemaphoreType.DMA,
        pltpu.SemaphoreType.DMA,
        pltpu.SemaphoreType.DMA,
        pltpu.SemaphoreType.DMA,
        pltpu.SemaphoreType.DMA,
        pltpu.SemaphoreType.DMA,
    ],
)(_sc_body)


@jax.jit
def kernel(etype, rid, att_rc, att_rp, W_type, W_rid, W_rc, b_rc, W_rp, b_rp):
    etype = etype.astype(jnp.int32)
    rid = rid.astype(jnp.int32)
    combo = ((W_type[:, None, :] + W_rid[None, :, :])
             .reshape(_NTYPE * _NRID, _HID) + b_rc + b_rp)
    wcat = jnp.concatenate([W_rc.T, W_rp.T], axis=0).reshape(-1)
    att_rp = att_rp.astype(jnp.float32)
    packed_i = (jnp.stack([etype, rid], axis=0)
                .reshape(2, _E // _NB, _NB)
                .transpose(1, 0, 2)
                .reshape(-1))
    packed_f = (jnp.stack([att_rc[:, 0], att_rc[:, 1],
                           att_rp[:, 0], att_rp[:, 1], att_rp[:, 2]], axis=0)
                .reshape(5, _E // _NB, _NB)
                .transpose(1, 0, 2)
                .reshape(-1))
    return _sc_call(combo, packed_i, packed_f, wcat)


# speedup vs baseline: 3.9275x; 1.2103x over previous
import functools

import jax
import jax.numpy as jnp
from jax import lax
from jax.experimental import pallas as pl
from jax.experimental.pallas import tpu as pltpu
from jax.experimental.pallas import tpu_sc as plsc

_E = 320000
_HID = 128
_NTYPE = 35
_NRID = 9
_NC = 2
_NS = 16
_NW = _NC * _NS
_PER_W = _E // _NW
_NB = 80
_CHUNKS = _PER_W // _NB
_L = 16
_PKI = 2 * _NB
_PKF = 5 * _NB


def _splat(vec, lane):
    return vec[jnp.full((_L,), lane, jnp.int32)]


def _sc_body(combo_h, pki_h, pkf_h, w_h, out_h,
             pki0, pki1, pkf0, pkf1, idx0, idx1, rows0, rows1, w_v,
             is0, is1, gs0, gs1, os0, os1):
    wid = lax.axis_index("s") * _NC + lax.axis_index("c")
    base = wid * _PER_W

    pltpu.sync_copy(w_h, w_v)
    pki = (pki0, pki1)
    pkf = (pkf0, pkf1)
    idxv = (idx0, idx1)
    rows = (rows0, rows1)
    isem = (is0, is1)
    gsem = (gs0, gs1)
    osem = (os0, os1)

    def in_copies(c, d):
        gc = wid * _CHUNKS + c
        return (
            pltpu.make_async_copy(
                pki_h.at[pl.ds(gc * _PKI, _PKI)], pki[d], isem[d]),
            pltpu.make_async_copy(
                pkf_h.at[pl.ds(gc * _PKF, _PKF)], pkf[d], isem[d]),
        )

    def in_start(c, d):
        for cp in in_copies(c, d):
            cp.start()

    def in_wait(c, d):
        for cp in in_copies(c, d):
            cp.wait()

    def gather_copy(d):
        return pltpu.make_async_copy(combo_h.at[idxv[d]], rows[d], gsem[d])

    def out_copy(c, d):
        cb = base + c * _NB
        return pltpu.make_async_copy(
            rows[d], out_h.at[pl.ds(cb, _NB)], osem[d])

    def compute_idx(d):
        for s in range(_NB // _L):
            et = pki[d][pl.ds(_L * s, _L)]
            ri = pki[d][pl.ds(_NB + _L * s, _L)]
            idxv[d][pl.ds(_L * s, _L)] = et * _NRID + ri

    def fma(d):
        def grp(gg, carry):
            av = [pkf[d][pl.ds(k * _NB + _L * gg, _L)] for k in range(5)]
            eb = gg * _L
            for h in range(2):
                wvh = [[w_v[pl.ds(128 * k + 64 * h + 16 * q, _L)]
                        for q in range(4)] for k in range(5)]
                for j in range(_L):
                    cf = [_splat(av[k], j) for k in range(5)]
                    for q in range(4):
                        col = 64 * h + 16 * q
                        r = rows[d][eb + j, pl.ds(col, _L)]
                        acc = (r + cf[0] * wvh[0][q] + cf[1] * wvh[1][q]
                               + cf[2] * wvh[2][q] + cf[3] * wvh[3][q]
                               + cf[4] * wvh[4][q])
                        rows[d][eb + j, pl.ds(col, _L)] = acc
            return carry
        lax.fori_loop(0, _NB // _L, grp, 0)

    def do_step(c, d, first=False, fire_gather=True, fire_in=True):
        dn = 1 - d
        if fire_gather:
            in_wait(c + 1, dn)
            compute_idx(dn)
            if not first:
                out_copy(c - 1, dn).wait()
            gather_copy(dn).start()
        gather_copy(d).wait()
        fma(d)
        out_copy(c, d).start()
        if fire_in:
            in_start(c + 2, d)

    in_start(0, 0)
    in_start(1, 1)
    in_wait(0, 0)
    compute_idx(0)
    gather_copy(0).start()

    do_step(0, 0, first=True)

    def pair(i, carry):
        c = 2 * i
        do_step(c, 0)
        do_step(c + 1, 1)
        return carry

    do_step(1, 1)
    lax.fori_loop(1, (_CHUNKS - 3) // 2, pair, 0)
    do_step(_CHUNKS - 3, 0)
    do_step(_CHUNKS - 2, 1, fire_in=False)
    do_step(_CHUNKS - 1, 0, fire_gather=False, fire_in=False)

    out_copy(_CHUNKS - 2, 1).wait()
    out_copy(_CHUNKS - 1, 0).wait()


_sc_call = functools.partial(
    pl.kernel,
    out_type=jax.ShapeDtypeStruct((_E, _HID), jnp.float32),
    mesh=plsc.VectorSubcoreMesh(
        core_axis_name="c", subcore_axis_name="s",
        num_cores=_NC, num_subcores=_NS),
    scratch_types=[
        pltpu.VMEM((_PKI,), jnp.int32),
        pltpu.VMEM((_PKI,), jnp.int32),
        pltpu.VMEM((_PKF,), jnp.float32),
        pltpu.VMEM((_PKF,), jnp.float32),
        pltpu.VMEM((_NB,), jnp.int32),
        pltpu.VMEM((_NB,), jnp.int32),
        pltpu.VMEM((_NB, _HID), jnp.float32),
        pltpu.VMEM((_NB, _HID), jnp.float32),
        pltpu.VMEM((5 * _HID,), jnp.float32),
        pltpu.SemaphoreType.DMA,
        pltpu.SemaphoreType.DMA,
        pltpu.SemaphoreType.DMA,
        pltpu.SemaphoreType.DMA,
        pltpu.SemaphoreType.DMA,
        pltpu.SemaphoreType.DMA,
    ],
)(_sc_body)


@jax.jit
def kernel(etype, rid, att_rc, att_rp, W_type, W_rid, W_rc, b_rc, W_rp, b_rp):
    etype = etype.astype(jnp.int32)
    rid = rid.astype(jnp.int32)
    combo = ((W_type[:, None, :] + W_rid[None, :, :])
             .reshape(_NTYPE * _NRID, _HID) + b_rc + b_rp)
    wcat = jnp.concatenate([W_rc.T, W_rp.T], axis=0).reshape(-1)
    att_rp = att_rp.astype(jnp.float32)
    packed_i = (jnp.stack([etype, rid], axis=0)
                .reshape(2, _E // _NB, _NB)
                .transpose(1, 0, 2)
                .reshape(-1))
    packed_f = (jnp.stack([att_rc[:, 0], att_rc[:, 1],
                           att_rp[:, 0], att_rp[:, 1], att_rp[:, 2]], axis=0)
                .reshape(5, _E // _NB, _NB)
                .transpose(1, 0, 2)
                .reshape(-1))
    return _sc_call(combo, packed_i, packed_f, wcat)
